# Initial kernel scaffold; baseline (speedup 1.0000x reference)
import jax
import jax.numpy as jnp
from jax.experimental import pallas as pl


def kernel(x, edge_index, edge_attr, W_enc, b_enc, t0, W1_0, b1_0, g1_0, be1_0, W2_0, b2_0, t1, W1_1, b1_1, g1_1, be1_1, W2_1, b2_1, n0_g, n0_b, n1_g, n1_b, W_lin, b_lin):
    raise NotImplementedError("write your pallas kernel here")


# speedup vs baseline: 3.5870x; 3.5870x over previous
import functools

import jax
import jax.numpy as jnp
from jax import lax
from jax.experimental import pallas as pl
from jax.experimental.pallas import tpu as pltpu
from jax.experimental.pallas import tpu_sc as plsc

F32 = jnp.float32
EPS = 1e-7
LN_EPS = 1e-5
DEN_EPS = 1e-16

NSUB = 16
LANES = 16
CH = 80


def _sc_aggr_kernel(N_PAD, E, H, gather_input):
    H2 = 2 * H
    EW = E // NSUB
    NCH = EW // CH
    NCHP = ((NCH + 7) // 8) * 8
    ROWS_W = N_PAD // NSUB

    mesh = plsc.VectorSubcoreMesh(core_axis_name="c", subcore_axis_name="s")

    out_type = [
        jax.ShapeDtypeStruct((E, H2), F32),
        jax.ShapeDtypeStruct((E, H2), F32),
        jax.ShapeDtypeStruct((E, H2), F32),
        jax.ShapeDtypeStruct((N_PAD, H2), F32),
        jax.ShapeDtypeStruct((N_PAD, H2), F32),
    ]
    scratch = [
        pltpu.VMEM((NCHP, CH), jnp.int32),
        pltpu.VMEM((NCHP, CH), jnp.int32),
        pltpu.VMEM((CH, H), F32),
        pltpu.VMEM((CH, H), F32),
        pltpu.VMEM((CH, H2), F32),
        pltpu.VMEM((CH, H2), F32),
        pltpu.VMEM((CH, H2), F32),
        pltpu.VMEM((CH, H2), F32),
        pltpu.VMEM((LANES,), F32),
        pltpu.VMEM_SHARED((N_PAD, H2), F32),
        pltpu.SemaphoreType.DMA,
        pltpu.SemaphoreType.DMA,
    ]

    @functools.partial(pl.kernel, out_type=out_type, mesh=mesh,
                       scratch_types=scratch,
                       compiler_params=pltpu.CompilerParams(
                           use_tc_tiling_on_sc=False))
    def body(h_hbm, row2d, col2d, tvec, xl0_hbm, denr_hbm, numr_hbm,
             den_hbm, num_hbm, rowv, colv, hr_v, hc_v, contrib, xlbuf,
             gbuf, zbuf, tv_v, acc, sem1, sem2):
        c = lax.axis_index("c")
        s = lax.axis_index("s")
        is0 = c == 0
        ebase = s * EW

        z16 = jnp.zeros((LANES,), F32)

        def zrow(i, carry):
            zbuf[i, pl.ds(0, LANES)] = z16
            zbuf[i, pl.ds(LANES, LANES)] = z16
            return carry

        lax.fori_loop(0, CH, zrow, 0, unroll=8)
        for z in range(ROWS_W // CH):
            pltpu.sync_copy(zbuf, acc.at[pl.ds(s * ROWS_W + z * CH, CH)])

        pltpu.sync_copy(row2d.at[pl.ds(s * NCHP, NCHP)], rowv)
        pltpu.sync_copy(col2d.at[pl.ds(s * NCHP, NCHP)], colv)
        pltpu.sync_copy(tvec, tv_v)
        tv = tv_v[...]
        epsv = jnp.full((LANES,), EPS, F32)

        plsc.subcore_barrier()

        def chunk(j, carry):
            if gather_input:
                cp1 = pltpu.async_copy(h_hbm.at[rowv.at[j]], hr_v, sem1)
                cp2 = pltpu.async_copy(h_hbm.at[colv.at[j]], hc_v, sem2)
                cp1.wait()
                cp2.wait()
            else:
                pltpu.sync_copy(h_hbm.at[pl.ds(ebase + j * CH, CH)], xlbuf)

            def edge(e, ecarry):
                if gather_input:
                    hr = hr_v[e]
                    hc = hc_v[e]
                    mr = jnp.maximum(hr, 0.0) + epsv
                    mc = jnp.maximum(hc, 0.0) + epsv
                    xlbuf[e, pl.ds(0, LANES)] = hr
                    xlbuf[e, pl.ds(LANES, LANES)] = hc
                else:
                    mr = xlbuf[e, pl.ds(0, LANES)] + epsv
                    mc = xlbuf[e, pl.ds(LANES, LANES)] + epsv
                er = jnp.exp(mr * tv)
                ec = jnp.exp(mc * tv)
                contrib[e, pl.ds(0, LANES)] = jnp.where(is0, er, mr * er)
                contrib[e, pl.ds(LANES, LANES)] = jnp.where(is0, ec, mc * ec)
                return ecarry

            lax.fori_loop(0, CH, edge, 0, unroll=4)
            pltpu.sync_copy(contrib, acc.at[colv.at[j]], add=True)
            if gather_input:
                @pl.when(is0)
                def _():
                    pltpu.sync_copy(
                        xlbuf, xl0_hbm.at[pl.ds(ebase + j * CH, CH)])
            return carry

        lax.fori_loop(0, NCH, chunk, 0)
        plsc.subcore_barrier()

        nslice = pl.ds(s * ROWS_W, ROWS_W)

        @pl.when(is0)
        def _():
            pltpu.sync_copy(acc.at[nslice], den_hbm.at[nslice])

        @pl.when(jnp.logical_not(is0))
        def _():
            pltpu.sync_copy(acc.at[nslice], num_hbm.at[nslice])

        plsc.subcore_barrier()

        @pl.when(is0)
        def _():
            def g(j, carry):
                pltpu.async_copy(den_hbm.at[rowv.at[j]], gbuf, sem1).wait()
                pltpu.sync_copy(gbuf, denr_hbm.at[pl.ds(ebase + j * CH, CH)])
                return carry
            lax.fori_loop(0, NCH, g, 0)

        @pl.when(jnp.logical_not(is0))
        def _():
            def g(j, carry):
                pltpu.async_copy(num_hbm.at[rowv.at[j]], gbuf, sem2).wait()
                pltpu.sync_copy(gbuf, numr_hbm.at[pl.ds(ebase + j * CH, CH)])
                return carry
            lax.fori_loop(0, NCH, g, 0)

    return body


def _layer_norm_block(z, g, b):
    mu = jnp.mean(z, axis=-1, keepdims=True)
    d = z - mu
    var = jnp.mean(d * d, axis=-1, keepdims=True)
    return d / jnp.sqrt(var + LN_EPS) * g + b


def _enc_body(x_ref, w_ref, b_ref, o_ref):
    o_ref[...] = (
        jnp.dot(x_ref[...], w_ref[...], preferred_element_type=F32)
        + b_ref[...]
    )


def _mid_body(xl0_ref, denr_ref, numr_ref, w1_ref, b1_ref, g1_ref, be1_ref,
              w2_ref, b2_ref, ng_ref, nb_ref, xl1_ref, hh_ref):
    aggr = numr_ref[...] / (denr_ref[...] + DEN_EPS)
    out = aggr + xl0_ref[...]
    z = jnp.dot(out, w1_ref[...], preferred_element_type=F32) + b1_ref[...]
    z = _layer_norm_block(z, g1_ref[...], be1_ref[...])
    z = jnp.maximum(z, 0.0)
    xl1 = jnp.dot(z, w2_ref[...], preferred_element_type=F32) + b2_ref[...]
    xl1_ref[...] = xl1
    hh_ref[...] = jnp.maximum(_layer_norm_block(xl1, ng_ref[...], nb_ref[...]),
                              0.0)


def _final_body(hh_ref, xl1_ref, denr_ref, numr_ref, w1_ref, b1_ref, g1_ref,
                be1_ref, w2_ref, b2_ref, n0g_ref, n0b_ref, wl_ref, bl_ref,
                y_ref):
    aggr = numr_ref[...] / (denr_ref[...] + DEN_EPS)
    out = aggr + hh_ref[...]
    z = jnp.dot(out, w1_ref[...], preferred_element_type=F32) + b1_ref[...]
    z = _layer_norm_block(z, g1_ref[...], be1_ref[...])
    z = jnp.maximum(z, 0.0)
    z = jnp.dot(z, w2_ref[...], preferred_element_type=F32) + b2_ref[...]
    xl2 = xl1_ref[...] + z
    q = jnp.maximum(_layer_norm_block(xl2, n0g_ref[...], n0b_ref[...]), 0.0)
    y_ref[...] = (
        jnp.dot(q, wl_ref[...], preferred_element_type=F32) + bl_ref[...]
    )


def kernel(x, edge_index, edge_attr, W_enc, b_enc, t0, W1_0, b1_0, g1_0,
           be1_0, W2_0, b2_0, t1, W1_1, b1_1, g1_1, be1_1, W2_1, b2_1,
           n0_g, n0_b, n1_g, n1_b, W_lin, b_lin):
    N, F_in = x.shape
    E = edge_index.shape[1]
    H = W_enc.shape[1]
    H2 = 2 * H
    Hm = W1_0.shape[1]
    F_out = W_lin.shape[1]

    N_PAD = ((N + NSUB * CH - 1) // (NSUB * CH)) * (NSUB * CH)

    nch = (E // NSUB) // CH
    nchp = ((nch + 7) // 8) * 8

    def pad_idx(v):
        v3 = v.astype(jnp.int32).reshape(NSUB, nch, CH)
        v3 = jnp.pad(v3, ((0, 0), (0, nchp - nch), (0, 0)))
        return v3.reshape(NSUB * nchp, CH)

    row2d = pad_idx(edge_index[0])
    col2d = pad_idx(edge_index[1])
    t0v = jnp.full((LANES,), t0, F32)
    t1v = jnp.full((LANES,), t1, F32)

    h = pl.pallas_call(
        _enc_body,
        out_shape=jax.ShapeDtypeStruct((N_PAD, H), F32),
        in_specs=[
            pl.BlockSpec((N_PAD, F_in), lambda: (0, 0)),
            pl.BlockSpec((F_in, H), lambda: (0, 0)),
            pl.BlockSpec((1, H), lambda: (0, 0)),
        ],
        out_specs=pl.BlockSpec((N_PAD, H), lambda: (0, 0)),
    )(jnp.pad(x, ((0, N_PAD - N), (0, 0))), W_enc, b_enc.reshape(1, H))

    sc0 = _sc_aggr_kernel(N_PAD, E, H, gather_input=True)
    xl0, den0r, num0r, _, _ = sc0(h, row2d, col2d, t0v)

    BE = 1600
    nblk = E // BE
    wspec = lambda shape: pl.BlockSpec(shape, lambda i: (0, 0))
    espec = pl.BlockSpec((BE, H2), lambda i: (i, 0))
    xl1, hh = pl.pallas_call(
        _mid_body,
        grid=(nblk,),
        out_shape=[
            jax.ShapeDtypeStruct((E, H2), F32),
            jax.ShapeDtypeStruct((E, H2), F32),
        ],
        in_specs=[
            espec, espec, espec,
            wspec((H2, Hm)), wspec((1, Hm)), wspec((1, Hm)), wspec((1, Hm)),
            wspec((Hm, H2)), wspec((1, H2)),
            wspec((1, H2)), wspec((1, H2)),
        ],
        out_specs=[espec, espec],
    )(xl0, den0r, num0r, W1_0, b1_0.reshape(1, Hm), g1_0.reshape(1, Hm),
      be1_0.reshape(1, Hm), W2_0, b2_0.reshape(1, H2), n1_g.reshape(1, H2),
      n1_b.reshape(1, H2))

    sc1 = _sc_aggr_kernel(N_PAD, E, H, gather_input=False)
    _, den1r, num1r, _, _ = sc1(hh, row2d, col2d, t1v)

    y = pl.pallas_call(
        _final_body,
        grid=(nblk,),
        out_shape=jax.ShapeDtypeStruct((E, F_out), F32),
        in_specs=[
            espec, espec, espec, espec,
            wspec((H2, Hm)), wspec((1, Hm)), wspec((1, Hm)), wspec((1, Hm)),
            wspec((Hm, H2)), wspec((1, H2)),
            wspec((1, H2)), wspec((1, H2)),
            wspec((H2, F_out)), wspec((1, F_out)),
        ],
        out_specs=pl.BlockSpec((BE, F_out), lambda i: (i, 0)),
    )(hh, xl1, den1r, num1r, W1_1, b1_1.reshape(1, Hm), g1_1.reshape(1, Hm),
      be1_1.reshape(1, Hm), W2_1, b2_1.reshape(1, H2), n0_g.reshape(1, H2),
      n0_b.reshape(1, H2), W_lin, b_lin.reshape(1, F_out))

    return y


# speedup vs baseline: 4.5514x; 1.2689x over previous
import functools

import jax
import jax.numpy as jnp
from jax import lax
from jax.experimental import pallas as pl
from jax.experimental.pallas import tpu as pltpu
from jax.experimental.pallas import tpu_sc as plsc

F32 = jnp.float32
EPS = 1e-7
LN_EPS = 1e-5
DEN_EPS = 1e-16

NSUB = 16
LANES = 16
CH = 80


def _sc_aggr_kernel(N_PAD, E, H, gather_input):
    H2 = 2 * H
    EW = E // NSUB
    NCH = EW // CH
    NCHP = ((NCH + 7) // 8) * 8
    ROWS_W = N_PAD // NSUB

    mesh = plsc.VectorSubcoreMesh(core_axis_name="c", subcore_axis_name="s")

    out_type = [
        jax.ShapeDtypeStruct((E, H2), F32),
        jax.ShapeDtypeStruct((E, H2), F32),
        jax.ShapeDtypeStruct((E, H2), F32),
        jax.ShapeDtypeStruct((N_PAD, H2), F32),
        jax.ShapeDtypeStruct((N_PAD, H2), F32),
    ]
    scratch = [
        pltpu.VMEM((NCHP, CH), jnp.int32),
        pltpu.VMEM((NCHP, CH), jnp.int32),
        pltpu.VMEM((2, CH, H), F32),
        pltpu.VMEM((2, CH, H), F32),
        pltpu.VMEM((2, CH, H2), F32),
        pltpu.VMEM((2, CH, H2), F32),
        pltpu.VMEM((2, CH, H2), F32),
        pltpu.VMEM((CH, H2), F32),
        pltpu.VMEM((LANES,), F32),
        pltpu.VMEM_SHARED((N_PAD, H2), F32),
        pltpu.SemaphoreType.DMA((2,)),
        pltpu.SemaphoreType.DMA((2,)),
        pltpu.SemaphoreType.DMA((2,)),
        pltpu.SemaphoreType.DMA((2,)),
    ]

    @functools.partial(pl.kernel, out_type=out_type, mesh=mesh,
                       scratch_types=scratch,
                       compiler_params=pltpu.CompilerParams(
                           use_tc_tiling_on_sc=False))
    def body(h_hbm, row2d, col2d, tvec, xl0_hbm, denr_hbm, numr_hbm,
             den_hbm, num_hbm, rowv, colv, hr_v, hc_v, contrib, xlbuf,
             gbuf, zbuf, tv_v, acc, semg, semc, sems, semo):
        c = lax.axis_index("c")
        s = lax.axis_index("s")
        is0 = c == 0
        ebase = s * EW

        z16 = jnp.zeros((LANES,), F32)

        def zrow(i, carry):
            zbuf[i, pl.ds(0, LANES)] = z16
            zbuf[i, pl.ds(LANES, LANES)] = z16
            return carry

        lax.fori_loop(0, CH, zrow, 0, unroll=8)
        for z in range(ROWS_W // CH):
            pltpu.sync_copy(zbuf, acc.at[pl.ds(s * ROWS_W + z * CH, CH)])

        pltpu.sync_copy(row2d.at[pl.ds(s * NCHP, NCHP)], rowv)
        pltpu.sync_copy(col2d.at[pl.ds(s * NCHP, NCHP)], colv)
        pltpu.sync_copy(tvec, tv_v)
        tv = tv_v[...]
        epsv = jnp.full((LANES,), EPS, F32)

        plsc.subcore_barrier()

        def issue_in(j, p):
            if gather_input:
                pltpu.async_copy(h_hbm.at[rowv.at[j]], hr_v.at[p], semg.at[p])
                pltpu.async_copy(h_hbm.at[colv.at[j]], hc_v.at[p], semc.at[p])
            else:
                pltpu.async_copy(h_hbm.at[pl.ds(ebase + j * CH, CH)],
                                 xlbuf.at[p], semg.at[p])

        def wait_in(j, p):
            if gather_input:
                pltpu.make_async_copy(h_hbm.at[rowv.at[j]], hr_v.at[p],
                                      semg.at[p]).wait()
                pltpu.make_async_copy(h_hbm.at[colv.at[j]], hc_v.at[p],
                                      semc.at[p]).wait()
            else:
                pltpu.make_async_copy(h_hbm.at[pl.ds(ebase + j * CH, CH)],
                                      xlbuf.at[p], semg.at[p]).wait()

        def wait_scat(j, p):
            pltpu.make_async_copy(contrib.at[p], acc.at[colv.at[j]],
                                  sems.at[p]).wait()
            if gather_input:
                @pl.when(is0)
                def _():
                    pltpu.make_async_copy(
                        xlbuf.at[p], xl0_hbm.at[pl.ds(ebase + j * CH, CH)],
                        semo.at[p]).wait()

        issue_in(0, 0)

        def chunk(j, carry):
            p = lax.rem(j, 2)
            q = 1 - p

            @pl.when(j + 1 < NCH)
            def _():
                issue_in(j + 1, q)

            wait_in(j, p)

            @pl.when(j >= 2)
            def _():
                wait_scat(j, p)

            def edge(e, ecarry):
                if gather_input:
                    hr = hr_v[p, e]
                    hc = hc_v[p, e]
                    mr = jnp.maximum(hr, 0.0) + epsv
                    mc = jnp.maximum(hc, 0.0) + epsv
                    xlbuf[p, e, pl.ds(0, LANES)] = hr
                    xlbuf[p, e, pl.ds(LANES, LANES)] = hc
                else:
                    mr = xlbuf[p, e, pl.ds(0, LANES)] + epsv
                    mc = xlbuf[p, e, pl.ds(LANES, LANES)] + epsv
                er = jnp.exp(mr * tv)
                ec = jnp.exp(mc * tv)
                contrib[p, e, pl.ds(0, LANES)] = jnp.where(is0, er, mr * er)
                contrib[p, e, pl.ds(LANES, LANES)] = jnp.where(is0, ec,
                                                               mc * ec)
                return ecarry

            lax.fori_loop(0, CH, edge, 0, unroll=4)
            pltpu.async_copy(contrib.at[p], acc.at[colv.at[j]], sems.at[p],
                             add=True)
            if gather_input:
                @pl.when(is0)
                def _():
                    pltpu.async_copy(
                        xlbuf.at[p], xl0_hbm.at[pl.ds(ebase + j * CH, CH)],
                        semo.at[p])
            return carry

        lax.fori_loop(0, NCH, chunk, 0)
        wait_scat(NCH - 2, (NCH - 2) % 2)
        wait_scat(NCH - 1, (NCH - 1) % 2)
        plsc.subcore_barrier()

        nslice = pl.ds(s * ROWS_W, ROWS_W)

        @pl.when(is0)
        def _():
            pltpu.sync_copy(acc.at[nslice], den_hbm.at[nslice])

        @pl.when(jnp.logical_not(is0))
        def _():
            pltpu.sync_copy(acc.at[nslice], num_hbm.at[nslice])

        plsc.subcore_barrier()

        def phase3(src_hbm, dst_hbm):
            def issue_g(j, p):
                pltpu.async_copy(src_hbm.at[rowv.at[j]], gbuf.at[p],
                                 semg.at[p])

            def wait_g(j, p):
                pltpu.make_async_copy(src_hbm.at[rowv.at[j]], gbuf.at[p],
                                      semg.at[p]).wait()

            def wait_out(j, p):
                pltpu.make_async_copy(
                    gbuf.at[p], dst_hbm.at[pl.ds(ebase + j * CH, CH)],
                    semo.at[p]).wait()

            issue_g(0, 0)

            def g(j, carry):
                p = lax.rem(j, 2)
                q = 1 - p

                @pl.when(j >= 1)
                def _():
                    wait_out(j - 1, q)

                @pl.when(j + 1 < NCH)
                def _():
                    issue_g(j + 1, q)

                wait_g(j, p)
                pltpu.async_copy(gbuf.at[p],
                                 dst_hbm.at[pl.ds(ebase + j * CH, CH)],
                                 semo.at[p])
                return carry

            lax.fori_loop(0, NCH, g, 0)
            wait_out(NCH - 1, (NCH - 1) % 2)

        @pl.when(is0)
        def _():
            phase3(den_hbm, denr_hbm)

        @pl.when(jnp.logical_not(is0))
        def _():
            phase3(num_hbm, numr_hbm)

    return body


def _layer_norm_block(z, g, b):
    mu = jnp.mean(z, axis=-1, keepdims=True)
    d = z - mu
    var = jnp.mean(d * d, axis=-1, keepdims=True)
    return d / jnp.sqrt(var + LN_EPS) * g + b


def _enc_body(x_ref, w_ref, b_ref, o_ref):
    o_ref[...] = (
        jnp.dot(x_ref[...], w_ref[...], preferred_element_type=F32)
        + b_ref[...]
    )


def _mid_body(xl0_ref, denr_ref, numr_ref, w1_ref, b1_ref, g1_ref, be1_ref,
              w2_ref, b2_ref, ng_ref, nb_ref, xl1_ref, hh_ref):
    aggr = numr_ref[...] / (denr_ref[...] + DEN_EPS)
    out = aggr + xl0_ref[...]
    z = jnp.dot(out, w1_ref[...], preferred_element_type=F32) + b1_ref[...]
    z = _layer_norm_block(z, g1_ref[...], be1_ref[...])
    z = jnp.maximum(z, 0.0)
    xl1 = jnp.dot(z, w2_ref[...], preferred_element_type=F32) + b2_ref[...]
    xl1_ref[...] = xl1
    hh_ref[...] = jnp.maximum(_layer_norm_block(xl1, ng_ref[...], nb_ref[...]),
                              0.0)


def _final_body(hh_ref, xl1_ref, denr_ref, numr_ref, w1_ref, b1_ref, g1_ref,
                be1_ref, w2_ref, b2_ref, n0g_ref, n0b_ref, wl_ref, bl_ref,
                y_ref):
    aggr = numr_ref[...] / (denr_ref[...] + DEN_EPS)
    out = aggr + hh_ref[...]
    z = jnp.dot(out, w1_ref[...], preferred_element_type=F32) + b1_ref[...]
    z = _layer_norm_block(z, g1_ref[...], be1_ref[...])
    z = jnp.maximum(z, 0.0)
    z = jnp.dot(z, w2_ref[...], preferred_element_type=F32) + b2_ref[...]
    xl2 = xl1_ref[...] + z
    q = jnp.maximum(_layer_norm_block(xl2, n0g_ref[...], n0b_ref[...]), 0.0)
    y_ref[...] = (
        jnp.dot(q, wl_ref[...], preferred_element_type=F32) + bl_ref[...]
    )


def kernel(x, edge_index, edge_attr, W_enc, b_enc, t0, W1_0, b1_0, g1_0,
           be1_0, W2_0, b2_0, t1, W1_1, b1_1, g1_1, be1_1, W2_1, b2_1,
           n0_g, n0_b, n1_g, n1_b, W_lin, b_lin):
    N, F_in = x.shape
    E = edge_index.shape[1]
    H = W_enc.shape[1]
    H2 = 2 * H
    Hm = W1_0.shape[1]
    F_out = W_lin.shape[1]

    N_PAD = ((N + NSUB * CH - 1) // (NSUB * CH)) * (NSUB * CH)

    nch = (E // NSUB) // CH
    nchp = ((nch + 7) // 8) * 8

    def pad_idx(v):
        v3 = v.astype(jnp.int32).reshape(NSUB, nch, CH)
        v3 = jnp.pad(v3, ((0, 0), (0, nchp - nch), (0, 0)))
        return v3.reshape(NSUB * nchp, CH)

    row2d = pad_idx(edge_index[0])
    col2d = pad_idx(edge_index[1])
    t0v = jnp.full((LANES,), t0, F32)
    t1v = jnp.full((LANES,), t1, F32)

    h = pl.pallas_call(
        _enc_body,
        out_shape=jax.ShapeDtypeStruct((N_PAD, H), F32),
        in_specs=[
            pl.BlockSpec((N_PAD, F_in), lambda: (0, 0)),
            pl.BlockSpec((F_in, H), lambda: (0, 0)),
            pl.BlockSpec((1, H), lambda: (0, 0)),
        ],
        out_specs=pl.BlockSpec((N_PAD, H), lambda: (0, 0)),
    )(jnp.pad(x, ((0, N_PAD - N), (0, 0))), W_enc, b_enc.reshape(1, H))

    sc0 = _sc_aggr_kernel(N_PAD, E, H, gather_input=True)
    xl0, den0r, num0r, _, _ = sc0(h, row2d, col2d, t0v)

    BE = 1600
    nblk = E // BE
    wspec = lambda shape: pl.BlockSpec(shape, lambda i: (0, 0))
    espec = pl.BlockSpec((BE, H2), lambda i: (i, 0))
    xl1, hh = pl.pallas_call(
        _mid_body,
        grid=(nblk,),
        out_shape=[
            jax.ShapeDtypeStruct((E, H2), F32),
            jax.ShapeDtypeStruct((E, H2), F32),
        ],
        in_specs=[
            espec, espec, espec,
            wspec((H2, Hm)), wspec((1, Hm)), wspec((1, Hm)), wspec((1, Hm)),
            wspec((Hm, H2)), wspec((1, H2)),
            wspec((1, H2)), wspec((1, H2)),
        ],
        out_specs=[espec, espec],
    )(xl0, den0r, num0r, W1_0, b1_0.reshape(1, Hm), g1_0.reshape(1, Hm),
      be1_0.reshape(1, Hm), W2_0, b2_0.reshape(1, H2), n1_g.reshape(1, H2),
      n1_b.reshape(1, H2))

    sc1 = _sc_aggr_kernel(N_PAD, E, H, gather_input=False)
    _, den1r, num1r, _, _ = sc1(hh, row2d, col2d, t1v)

    y = pl.pallas_call(
        _final_body,
        grid=(nblk,),
        out_shape=jax.ShapeDtypeStruct((E, F_out), F32),
        in_specs=[
            espec, espec, espec, espec,
            wspec((H2, Hm)), wspec((1, Hm)), wspec((1, Hm)), wspec((1, Hm)),
            wspec((Hm, H2)), wspec((1, H2)),
            wspec((1, H2)), wspec((1, H2)),
            wspec((H2, F_out)), wspec((1, F_out)),
        ],
        out_specs=pl.BlockSpec((BE, F_out), lambda i: (i, 0)),
    )(hh, xl1, den1r, num1r, W1_1, b1_1.reshape(1, Hm), g1_1.reshape(1, Hm),
      be1_1.reshape(1, Hm), W2_1, b2_1.reshape(1, H2), n0_g.reshape(1, H2),
      n0_b.reshape(1, H2), W_lin, b_lin.reshape(1, F_out))

    return y


# speedup vs baseline: 4.7252x; 1.0382x over previous
import functools

import jax
import jax.numpy as jnp
from jax import lax
from jax.experimental import pallas as pl
from jax.experimental.pallas import tpu as pltpu
from jax.experimental.pallas import tpu_sc as plsc

F32 = jnp.float32
EPS = 1e-7
LN_EPS = 1e-5
DEN_EPS = 1e-16

NSUB = 16
LANES = 16
CH = 80


def _sc_aggr_kernel(N_PAD, E, H, gather_input):
    H2 = 2 * H
    H4 = 4 * H
    EW = E // NSUB
    NCH = EW // CH
    NCHP = ((NCH + 7) // 8) * 8
    ROWS_W = N_PAD // NSUB

    mesh = plsc.VectorSubcoreMesh(core_axis_name="c", subcore_axis_name="s",
                                  num_cores=1)

    out_type = [
        jax.ShapeDtypeStruct((E, H2), F32),
        jax.ShapeDtypeStruct((E, H4), F32),
        jax.ShapeDtypeStruct((N_PAD, H4), F32),
    ]
    scratch = [
        pltpu.VMEM((NCHP, CH), jnp.int32),
        pltpu.VMEM((NCHP, CH), jnp.int32),
        pltpu.VMEM((2, CH, H), F32),
        pltpu.VMEM((2, CH, H), F32),
        pltpu.VMEM((2, CH, H4), F32),
        pltpu.VMEM((2, CH, H2), F32),
        pltpu.VMEM((2, CH, H4), F32),
        pltpu.VMEM((CH, H4), F32),
        pltpu.VMEM((LANES,), F32),
        pltpu.VMEM_SHARED((N_PAD, H4), F32),
        pltpu.SemaphoreType.DMA((2,)),
        pltpu.SemaphoreType.DMA((2,)),
        pltpu.SemaphoreType.DMA((2,)),
        pltpu.SemaphoreType.DMA((2,)),
    ]

    @functools.partial(pl.kernel, out_type=out_type, mesh=mesh,
                       scratch_types=scratch,
                       compiler_params=pltpu.CompilerParams(
                           use_tc_tiling_on_sc=False))
    def body(h_hbm, row2d, col2d, tvec, xl0_hbm, dnr_hbm, dn_hbm,
             rowv, colv, hr_v, hc_v, contrib, xlbuf,
             gbuf, zbuf, tv_v, acc, semg, semc, sems, semo):
        s = lax.axis_index("s")
        ebase = s * EW

        z16 = jnp.zeros((LANES,), F32)

        def zrow(i, carry):
            for k in range(4):
                zbuf[i, pl.ds(k * LANES, LANES)] = z16
            return carry

        lax.fori_loop(0, CH, zrow, 0, unroll=8)
        for z in range(ROWS_W // CH):
            pltpu.sync_copy(zbuf, acc.at[pl.ds(s * ROWS_W + z * CH, CH)])

        pltpu.sync_copy(row2d.at[pl.ds(s * NCHP, NCHP)], rowv)
        pltpu.sync_copy(col2d.at[pl.ds(s * NCHP, NCHP)], colv)
        pltpu.sync_copy(tvec, tv_v)
        tv = tv_v[...]
        epsv = jnp.full((LANES,), EPS, F32)

        plsc.subcore_barrier()

        def issue_in(j, p):
            if gather_input:
                pltpu.async_copy(h_hbm.at[rowv.at[j]], hr_v.at[p], semg.at[p])
                pltpu.async_copy(h_hbm.at[colv.at[j]], hc_v.at[p], semc.at[p])
            else:
                pltpu.async_copy(h_hbm.at[pl.ds(ebase + j * CH, CH)],
                                 xlbuf.at[p], semg.at[p])

        def wait_in(j, p):
            if gather_input:
                pltpu.make_async_copy(h_hbm.at[rowv.at[j]], hr_v.at[p],
                                      semg.at[p]).wait()
                pltpu.make_async_copy(h_hbm.at[colv.at[j]], hc_v.at[p],
                                      semc.at[p]).wait()
            else:
                pltpu.make_async_copy(h_hbm.at[pl.ds(ebase + j * CH, CH)],
                                      xlbuf.at[p], semg.at[p]).wait()

        def wait_scat(j, p):
            pltpu.make_async_copy(contrib.at[p], acc.at[colv.at[j]],
                                  sems.at[p]).wait()
            if gather_input:
                pltpu.make_async_copy(
                    xlbuf.at[p], xl0_hbm.at[pl.ds(ebase + j * CH, CH)],
                    semo.at[p]).wait()

        issue_in(0, 0)

        def chunk(j, carry):
            p = lax.rem(j, 2)
            q = 1 - p

            @pl.when(j + 1 < NCH)
            def _():
                issue_in(j + 1, q)

            wait_in(j, p)

            @pl.when(j >= 2)
            def _():
                wait_scat(j, p)

            def edge(e, ecarry):
                if gather_input:
                    hr = hr_v[p, e]
                    hc = hc_v[p, e]
                    mr = jnp.maximum(hr, 0.0) + epsv
                    mc = jnp.maximum(hc, 0.0) + epsv
                    xlbuf[p, e, pl.ds(0, LANES)] = hr
                    xlbuf[p, e, pl.ds(LANES, LANES)] = hc
                else:
                    mr = xlbuf[p, e, pl.ds(0, LANES)] + epsv
                    mc = xlbuf[p, e, pl.ds(LANES, LANES)] + epsv
                er = jnp.exp(mr * tv)
                ec = jnp.exp(mc * tv)
                contrib[p, e, pl.ds(0, LANES)] = er
                contrib[p, e, pl.ds(LANES, LANES)] = ec
                contrib[p, e, pl.ds(2 * LANES, LANES)] = mr * er
                contrib[p, e, pl.ds(3 * LANES, LANES)] = mc * ec
                return ecarry

            lax.fori_loop(0, CH, edge, 0, unroll=4)
            pltpu.async_copy(contrib.at[p], acc.at[colv.at[j]], sems.at[p],
                             add=True)
            if gather_input:
                pltpu.async_copy(
                    xlbuf.at[p], xl0_hbm.at[pl.ds(ebase + j * CH, CH)],
                    semo.at[p])
            return carry

        lax.fori_loop(0, NCH, chunk, 0)
        wait_scat(NCH - 2, (NCH - 2) % 2)
        wait_scat(NCH - 1, (NCH - 1) % 2)
        plsc.subcore_barrier()

        nslice = pl.ds(s * ROWS_W, ROWS_W)
        pltpu.sync_copy(acc.at[nslice], dn_hbm.at[nslice])
        plsc.subcore_barrier()

        def issue_g(j, p):
            pltpu.async_copy(dn_hbm.at[rowv.at[j]], gbuf.at[p], semg.at[p])

        def wait_g(j, p):
            pltpu.make_async_copy(dn_hbm.at[rowv.at[j]], gbuf.at[p],
                                  semg.at[p]).wait()

        def wait_out(j, p):
            pltpu.make_async_copy(
                gbuf.at[p], dnr_hbm.at[pl.ds(ebase + j * CH, CH)],
                semo.at[p]).wait()

        issue_g(0, 0)

        def g(j, carry):
            p = lax.rem(j, 2)
            q = 1 - p

            @pl.when(j >= 1)
            def _():
                wait_out(j - 1, q)

            @pl.when(j + 1 < NCH)
            def _():
                issue_g(j + 1, q)

            wait_g(j, p)
            pltpu.async_copy(gbuf.at[p],
                             dnr_hbm.at[pl.ds(ebase + j * CH, CH)],
                             semo.at[p])
            return carry

        lax.fori_loop(0, NCH, g, 0)
        wait_out(NCH - 1, (NCH - 1) % 2)

    return body


def _layer_norm_block(z, g, b):
    mu = jnp.mean(z, axis=-1, keepdims=True)
    d = z - mu
    var = jnp.mean(d * d, axis=-1, keepdims=True)
    return d / jnp.sqrt(var + LN_EPS) * g + b


def _enc_body(x_ref, w_ref, b_ref, o_ref):
    o_ref[...] = (
        jnp.dot(x_ref[...], w_ref[...], preferred_element_type=F32)
        + b_ref[...]
    )


def _mid_body(xl0_ref, dnr_ref, w1_ref, b1_ref, g1_ref, be1_ref,
              w2_ref, b2_ref, ng_ref, nb_ref, xl1_ref, hh_ref):
    H2 = xl0_ref.shape[1]
    dnr = dnr_ref[...]
    aggr = dnr[:, H2:] / (dnr[:, :H2] + DEN_EPS)
    out = aggr + xl0_ref[...]
    z = jnp.dot(out, w1_ref[...], preferred_element_type=F32) + b1_ref[...]
    z = _layer_norm_block(z, g1_ref[...], be1_ref[...])
    z = jnp.maximum(z, 0.0)
    xl1 = jnp.dot(z, w2_ref[...], preferred_element_type=F32) + b2_ref[...]
    xl1_ref[...] = xl1
    hh_ref[...] = jnp.maximum(_layer_norm_block(xl1, ng_ref[...], nb_ref[...]),
                              0.0)


def _final_body(hh_ref, xl1_ref, dnr_ref, w1_ref, b1_ref, g1_ref,
                be1_ref, w2_ref, b2_ref, n0g_ref, n0b_ref, wl_ref, bl_ref,
                y_ref):
    H2 = hh_ref.shape[1]
    dnr = dnr_ref[...]
    aggr = dnr[:, H2:] / (dnr[:, :H2] + DEN_EPS)
    out = aggr + hh_ref[...]
    z = jnp.dot(out, w1_ref[...], preferred_element_type=F32) + b1_ref[...]
    z = _layer_norm_block(z, g1_ref[...], be1_ref[...])
    z = jnp.maximum(z, 0.0)
    z = jnp.dot(z, w2_ref[...], preferred_element_type=F32) + b2_ref[...]
    xl2 = xl1_ref[...] + z
    q = jnp.maximum(_layer_norm_block(xl2, n0g_ref[...], n0b_ref[...]), 0.0)
    y_ref[...] = (
        jnp.dot(q, wl_ref[...], preferred_element_type=F32) + bl_ref[...]
    )


def kernel(x, edge_index, edge_attr, W_enc, b_enc, t0, W1_0, b1_0, g1_0,
           be1_0, W2_0, b2_0, t1, W1_1, b1_1, g1_1, be1_1, W2_1, b2_1,
           n0_g, n0_b, n1_g, n1_b, W_lin, b_lin):
    N, F_in = x.shape
    E = edge_index.shape[1]
    H = W_enc.shape[1]
    H2 = 2 * H
    H4 = 4 * H
    Hm = W1_0.shape[1]
    F_out = W_lin.shape[1]

    N_PAD = ((N + NSUB * CH - 1) // (NSUB * CH)) * (NSUB * CH)

    nch = (E // NSUB) // CH
    nchp = ((nch + 7) // 8) * 8

    def pad_idx(v):
        v3 = v.astype(jnp.int32).reshape(NSUB, nch, CH)
        v3 = jnp.pad(v3, ((0, 0), (0, nchp - nch), (0, 0)))
        return v3.reshape(NSUB * nchp, CH)

    row2d = pad_idx(edge_index[0])
    col2d = pad_idx(edge_index[1])
    t0v = jnp.full((LANES,), t0, F32)
    t1v = jnp.full((LANES,), t1, F32)

    h = pl.pallas_call(
        _enc_body,
        out_shape=jax.ShapeDtypeStruct((N_PAD, H), F32),
        in_specs=[
            pl.BlockSpec((N_PAD, F_in), lambda: (0, 0)),
            pl.BlockSpec((F_in, H), lambda: (0, 0)),
            pl.BlockSpec((1, H), lambda: (0, 0)),
        ],
        out_specs=pl.BlockSpec((N_PAD, H), lambda: (0, 0)),
    )(jnp.pad(x, ((0, N_PAD - N), (0, 0))), W_enc, b_enc.reshape(1, H))

    sc0 = _sc_aggr_kernel(N_PAD, E, H, gather_input=True)
    xl0, dnr0, _ = sc0(h, row2d, col2d, t0v)

    BE = 1600
    nblk = E // BE
    wspec = lambda shape: pl.BlockSpec(shape, lambda i: (0, 0))
    espec = pl.BlockSpec((BE, H2), lambda i: (i, 0))
    dspec = pl.BlockSpec((BE, H4), lambda i: (i, 0))
    xl1, hh = pl.pallas_call(
        _mid_body,
        grid=(nblk,),
        out_shape=[
            jax.ShapeDtypeStruct((E, H2), F32),
            jax.ShapeDtypeStruct((E, H2), F32),
        ],
        in_specs=[
            espec, dspec,
            wspec((H2, Hm)), wspec((1, Hm)), wspec((1, Hm)), wspec((1, Hm)),
            wspec((Hm, H2)), wspec((1, H2)),
            wspec((1, H2)), wspec((1, H2)),
        ],
        out_specs=[espec, espec],
    )(xl0, dnr0, W1_0, b1_0.reshape(1, Hm), g1_0.reshape(1, Hm),
      be1_0.reshape(1, Hm), W2_0, b2_0.reshape(1, H2), n1_g.reshape(1, H2),
      n1_b.reshape(1, H2))

    sc1 = _sc_aggr_kernel(N_PAD, E, H, gather_input=False)
    _, dnr1, _ = sc1(hh, row2d, col2d, t1v)

    y = pl.pallas_call(
        _final_body,
        grid=(nblk,),
        out_shape=jax.ShapeDtypeStruct((E, F_out), F32),
        in_specs=[
            espec, espec, dspec,
            wspec((H2, Hm)), wspec((1, Hm)), wspec((1, Hm)), wspec((1, Hm)),
            wspec((Hm, H2)), wspec((1, H2)),
            wspec((1, H2)), wspec((1, H2)),
            wspec((H2, F_out)), wspec((1, F_out)),
        ],
        out_specs=pl.BlockSpec((BE, F_out), lambda i: (i, 0)),
    )(hh, xl1, dnr1, W1_1, b1_1.reshape(1, Hm), g1_1.reshape(1, Hm),
      be1_1.reshape(1, Hm), W2_1, b2_1.reshape(1, H2), n0_g.reshape(1, H2),
      n0_b.reshape(1, H2), W_lin, b_lin.reshape(1, F_out))

    return y
